# baseline (speedup 1.0000x reference)
import jax
import jax.numpy as jnp
from jax.experimental import pallas as pl
from jax.experimental.pallas import tpu as pltpu

_CHUNKS = (8, 6, 8, 6, 8, 6, 8, 6, 8)
_SLOTS = (8, 6)


def _copy_body(x_hbm, o_hbm, buf_a, buf_b, rsem, wsem):
    nch = len(_CHUNKS)
    offs = [sum(_CHUNKS[:i]) for i in range(nch)]
    bufs = (buf_a, buf_b)

    def read(i):
        b = i % 2
        return pltpu.make_async_copy(
            x_hbm.at[pl.ds(offs[i], _CHUNKS[i])],
            bufs[b].at[pl.ds(0, _CHUNKS[i])],
            rsem.at[b],
        )

    def write(i):
        b = i % 2
        return pltpu.make_async_copy(
            bufs[b].at[pl.ds(0, _CHUNKS[i])],
            o_hbm.at[pl.ds(offs[i], _CHUNKS[i])],
            wsem.at[b],
        )

    for i in range(nch):
        if i >= 2:
            write(i - 2).wait()
        read(i).start()
        if i >= 1:
            read(i - 1).wait()
            write(i - 1).start()
    read(nch - 1).wait()
    write(nch - 1).start()
    write(nch - 2).wait()
    write(nch - 1).wait()


def kernel(x):
    b, n, f = x.shape
    return pl.pallas_call(
        _copy_body,
        out_shape=jax.ShapeDtypeStruct(x.shape, x.dtype),
        in_specs=[pl.BlockSpec(memory_space=pltpu.MemorySpace.HBM)],
        out_specs=pl.BlockSpec(memory_space=pltpu.MemorySpace.HBM),
        scratch_shapes=[
            pltpu.VMEM((_SLOTS[0], n, f), x.dtype),
            pltpu.VMEM((_SLOTS[1], n, f), x.dtype),
            pltpu.SemaphoreType.DMA((2,)),
            pltpu.SemaphoreType.DMA((2,)),
        ],
    )(x)


# speedup vs baseline: 1.0017x; 1.0017x over previous
import jax
import jax.numpy as jnp
from jax.experimental import pallas as pl
from jax.experimental.pallas import tpu as pltpu

_CHUNKS = (2, 6, 6, 6, 6, 6, 6, 6, 6, 6, 6, 2)
_SLOTS = (6, 6)


def _copy_body(x_hbm, o_hbm, buf_a, buf_b, rsem, wsem):
    nch = len(_CHUNKS)
    offs = [sum(_CHUNKS[:i]) for i in range(nch)]
    bufs = (buf_a, buf_b)

    def read(i):
        b = i % 2
        return pltpu.make_async_copy(
            x_hbm.at[pl.ds(offs[i], _CHUNKS[i])],
            bufs[b].at[pl.ds(0, _CHUNKS[i])],
            rsem.at[b],
        )

    def write(i):
        b = i % 2
        return pltpu.make_async_copy(
            bufs[b].at[pl.ds(0, _CHUNKS[i])],
            o_hbm.at[pl.ds(offs[i], _CHUNKS[i])],
            wsem.at[b],
        )

    for i in range(nch):
        if i >= 2:
            write(i - 2).wait()
        read(i).start()
        if i >= 1:
            read(i - 1).wait()
            write(i - 1).start()
    read(nch - 1).wait()
    write(nch - 1).start()
    write(nch - 2).wait()
    write(nch - 1).wait()


def kernel(x):
    b, n, f = x.shape
    return pl.pallas_call(
        _copy_body,
        out_shape=jax.ShapeDtypeStruct(x.shape, x.dtype),
        in_specs=[pl.BlockSpec(memory_space=pltpu.MemorySpace.HBM)],
        out_specs=pl.BlockSpec(memory_space=pltpu.MemorySpace.HBM),
        scratch_shapes=[
            pltpu.VMEM((_SLOTS[0], n, f), x.dtype),
            pltpu.VMEM((_SLOTS[1], n, f), x.dtype),
            pltpu.SemaphoreType.DMA((2,)),
            pltpu.SemaphoreType.DMA((2,)),
        ],
    )(x)
